# baseline (speedup 1.0000x reference)
import functools

import jax
import jax.numpy as jnp
from jax import lax
from jax.experimental import pallas as pl
from jax.experimental.pallas import tpu as pltpu
from jax.experimental.pallas import tpu_sc as plsc

N, P3, W3, CKV = 8, 49, 64, 384
TOPK = 4
D = W3 * CKV
ROWS = N * P3
B = N * P3 * TOPK
NC, NS = 2, 16
NW = NC * NS
RPW = B // NW
NSEM = 8

_mesh = plsc.VectorSubcoreMesh(core_axis_name="c", subcore_axis_name="s")


@functools.partial(
    pl.kernel,
    mesh=_mesh,
    out_type=jax.ShapeDtypeStruct((B, D), jnp.float32),
    scratch_types=[
        pltpu.VMEM((64,), jnp.int32),
    ]
    + [pltpu.SemaphoreType.DMA for _ in range(NSEM)],
)
def _sc_gather(gidx_hbm, table_hbm, out_hbm, idx_v, *sems):
    wid = lax.axis_index("s") * NC + lax.axis_index("c")
    base = wid * RPW
    pltpu.sync_copy(gidx_hbm.at[wid], idx_v)

    def fire(i, k):
        row = idx_v[pl.ds(i, 16)][0]
        pltpu.async_copy(
            table_hbm.at[pl.ds(row, 1)],
            out_hbm.at[pl.ds(base + i, 1)],
            sems[k],
        )

    def drain(i, k):
        pltpu.make_async_copy(
            table_hbm.at[pl.ds(0, 1)],
            out_hbm.at[pl.ds(base + i, 1)],
            sems[k],
        ).wait()

    for k in range(NSEM):
        fire(k, k)

    def body(i, carry):
        k_cycle = lax.rem(i, NSEM)
        for k in range(NSEM):

            @pl.when(k_cycle == k)
            def _():
                drain(i, k)
                fire(i + NSEM, k)

        return carry

    lax.fori_loop(0, RPW - NSEM, body, 0)
    for i in range(RPW - NSEM, RPW):
        drain(i, i % NSEM)


def kernel(r_idx, qkv):
    gidx = (
        jnp.arange(N, dtype=jnp.int32)[:, None, None] * P3
        + r_idx.astype(jnp.int32)
    ).reshape(NW, RPW)
    gidx = jnp.pad(gidx, ((0, 0), (0, 64 - RPW)))
    table = qkv.reshape(ROWS, D)
    out = _sc_gather(gidx, table)
    return out.reshape(N, P3, TOPK, W3, CKV)


# speedup vs baseline: 15.9093x; 15.9093x over previous
import functools

import jax
import jax.numpy as jnp
from jax import lax
from jax.experimental import pallas as pl
from jax.experimental.pallas import tpu as pltpu
from jax.experimental.pallas import tpu_sc as plsc

N, P3, W3, CKV = 8, 49, 64, 384
TOPK = 4
D = W3 * CKV
ROWS = N * P3
B = N * P3 * TOPK
NC, NS = 2, 16
NW = NC * NS
RPW = B // NW

_mesh = plsc.VectorSubcoreMesh(core_axis_name="c", subcore_axis_name="s")


NBUF = 4


@functools.partial(
    pl.kernel,
    mesh=_mesh,
    out_type=jax.ShapeDtypeStruct((B, D), jnp.float32),
    scratch_types=[
        pltpu.VMEM((RPW, 1), jnp.int32),
    ]
    + [pltpu.VMEM((1, D), jnp.float32) for _ in range(NBUF)]
    + [pltpu.SemaphoreType.DMA for _ in range(2 * NBUF)],
)
def _sc_gather(gidx_hbm, table_hbm, out_hbm, idx_v, *scr):
    bufs = scr[:NBUF]
    gsems = scr[NBUF : 2 * NBUF]
    wsems = scr[2 * NBUF :]
    wid = lax.axis_index("s") * NC + lax.axis_index("c")
    base = wid * RPW
    pltpu.sync_copy(gidx_hbm.at[wid], idx_v)

    def fire_gather(i, slot):
        pltpu.async_copy(table_hbm.at[idx_v.at[i]], bufs[slot], gsems[slot])

    def wait_gather(i, slot):
        pltpu.make_async_copy(
            table_hbm.at[idx_v.at[i]], bufs[slot], gsems[slot]
        ).wait()

    def fire_write(i, slot):
        pltpu.async_copy(bufs[slot], out_hbm.at[pl.ds(base + i, 1)], wsems[slot])

    def wait_write(i, slot):
        pltpu.make_async_copy(
            bufs[slot], out_hbm.at[pl.ds(base + i, 1)], wsems[slot]
        ).wait()

    fire_gather(0, 0)
    fire_gather(1, 1)
    fire_gather(2, 2); wait_gather(0, 0); fire_write(0, 0)
    fire_gather(3, 3); wait_gather(1, 1); fire_write(1, 1)
    wait_write(0, 0); fire_gather(4, 0); wait_gather(2, 2); fire_write(2, 2)
    wait_write(1, 1); fire_gather(5, 1); wait_gather(3, 3); fire_write(3, 3)

    def body(g, carry):
        for k in range(4):
            i = 4 * g + k
            fslot = (k + 2) % 4
            wait_write(i - 2, fslot)
            fire_gather(i + 2, fslot)
            wait_gather(i, k)
            fire_write(i, k)
        return carry

    lax.fori_loop(1, 11, body, 0)
    wait_write(42, 2); fire_gather(46, 2); wait_gather(44, 0); fire_write(44, 0)
    wait_write(43, 3); fire_gather(47, 3); wait_gather(45, 1); fire_write(45, 1)
    wait_write(44, 0); fire_gather(48, 0); wait_gather(46, 2); fire_write(46, 2)
    wait_gather(47, 3); fire_write(47, 3)
    wait_gather(48, 0); fire_write(48, 0)
    wait_write(45, 1)
    wait_write(46, 2)
    wait_write(47, 3)
    wait_write(48, 0)


def kernel(r_idx, qkv):
    gidx = (
        jnp.arange(N, dtype=jnp.int32)[:, None, None] * P3
        + r_idx.astype(jnp.int32)
    ).reshape(NW, RPW, 1)
    table = qkv.reshape(ROWS, D)
    out = _sc_gather(gidx, table)
    return out.reshape(N, P3, TOPK, W3, CKV)


# speedup vs baseline: 38.9413x; 2.4477x over previous
import functools

import jax
import jax.numpy as jnp
from jax import lax
from jax.experimental import pallas as pl
from jax.experimental.pallas import tpu as pltpu
from jax.experimental.pallas import tpu_sc as plsc

N, P3, W3, CKV = 8, 49, 64, 384
TOPK = 4
D = W3 * CKV
ROWS = N * P3
B = N * P3 * TOPK
NC, NS = 2, 16
NW = NC * NS
RPW = B // NW

_mesh = plsc.VectorSubcoreMesh(core_axis_name="c", subcore_axis_name="s")


NBUF = 4


@functools.partial(
    pl.kernel,
    mesh=_mesh,
    out_type=jax.ShapeDtypeStruct((B, W3, CKV), jnp.float32),
    scratch_types=[
        pltpu.VMEM((RPW, 1), jnp.int32),
    ]
    + [pltpu.VMEM((1, W3, CKV), jnp.float32) for _ in range(NBUF)]
    + [pltpu.SemaphoreType.DMA for _ in range(2 * NBUF)],
)
def _sc_gather(gidx_hbm, table_hbm, out_hbm, idx_v, *scr):
    bufs = scr[:NBUF]
    gsems = scr[NBUF : 2 * NBUF]
    wsems = scr[2 * NBUF :]
    wid = lax.axis_index("s") * NC + lax.axis_index("c")
    base = wid * RPW
    pltpu.sync_copy(gidx_hbm.at[wid], idx_v)

    def fire_gather(i, slot):
        pltpu.async_copy(table_hbm.at[idx_v.at[i]], bufs[slot], gsems[slot])

    def wait_gather(i, slot):
        pltpu.make_async_copy(
            table_hbm.at[idx_v.at[i]], bufs[slot], gsems[slot]
        ).wait()

    def fire_write(i, slot):
        pltpu.async_copy(bufs[slot], out_hbm.at[pl.ds(base + i, 1)], wsems[slot])

    def wait_write(i, slot):
        pltpu.make_async_copy(
            bufs[slot], out_hbm.at[pl.ds(base + i, 1)], wsems[slot]
        ).wait()

    fire_gather(0, 0)
    fire_gather(1, 1)
    fire_gather(2, 2); wait_gather(0, 0); fire_write(0, 0)
    fire_gather(3, 3); wait_gather(1, 1); fire_write(1, 1)
    wait_write(0, 0); fire_gather(4, 0); wait_gather(2, 2); fire_write(2, 2)
    wait_write(1, 1); fire_gather(5, 1); wait_gather(3, 3); fire_write(3, 3)

    def body(g, carry):
        for k in range(4):
            i = 4 * g + k
            fslot = (k + 2) % 4
            wait_write(i - 2, fslot)
            fire_gather(i + 2, fslot)
            wait_gather(i, k)
            fire_write(i, k)
        return carry

    lax.fori_loop(1, 11, body, 0)
    wait_write(42, 2); fire_gather(46, 2); wait_gather(44, 0); fire_write(44, 0)
    wait_write(43, 3); fire_gather(47, 3); wait_gather(45, 1); fire_write(45, 1)
    wait_write(44, 0); fire_gather(48, 0); wait_gather(46, 2); fire_write(46, 2)
    wait_gather(47, 3); fire_write(47, 3)
    wait_gather(48, 0); fire_write(48, 0)
    wait_write(45, 1)
    wait_write(46, 2)
    wait_write(47, 3)
    wait_write(48, 0)


def kernel(r_idx, qkv):
    gidx = (
        jnp.arange(N, dtype=jnp.int32)[:, None, None] * P3
        + r_idx.astype(jnp.int32)
    ).reshape(NW, RPW, 1)
    table = qkv.reshape(ROWS, W3, CKV)
    out = _sc_gather(gidx, table)
    return out.reshape(N, P3, TOPK, W3, CKV)


# speedup vs baseline: 38.9629x; 1.0006x over previous
import functools

import jax
import jax.numpy as jnp
from jax import lax
from jax.experimental import pallas as pl
from jax.experimental.pallas import tpu as pltpu
from jax.experimental.pallas import tpu_sc as plsc

N, P3, W3, CKV = 8, 49, 64, 384
TOPK = 4
ROWS = N * P3
B = N * P3 * TOPK
NC, NS = 2, 16
NW = NC * NS
RPW = B // NW
NBUF = 5
AHEAD = 2

_mesh = plsc.VectorSubcoreMesh(core_axis_name="c", subcore_axis_name="s")


@functools.partial(
    pl.kernel,
    mesh=_mesh,
    out_type=jax.ShapeDtypeStruct((B, W3, CKV), jnp.float32),
    scratch_types=[
        pltpu.VMEM((RPW, 1), jnp.int32),
    ]
    + [pltpu.VMEM((1, W3, CKV), jnp.float32) for _ in range(NBUF)]
    + [pltpu.SemaphoreType.DMA for _ in range(2 * NBUF)],
)
def _sc_gather(gidx_hbm, table_hbm, out_hbm, idx_v, *scr):
    bufs = scr[:NBUF]
    gsems = scr[NBUF : 2 * NBUF]
    wsems = scr[2 * NBUF :]
    wid = lax.axis_index("s") * NC + lax.axis_index("c")
    base = wid * RPW
    pltpu.sync_copy(gidx_hbm.at[wid], idx_v)

    def fire_gather(i, slot):
        pltpu.async_copy(table_hbm.at[idx_v.at[i]], bufs[slot], gsems[slot])

    def wait_gather(i, slot):
        pltpu.make_async_copy(
            table_hbm.at[idx_v.at[i]], bufs[slot], gsems[slot]
        ).wait()

    def fire_write(i, slot):
        pltpu.async_copy(bufs[slot], out_hbm.at[pl.ds(base + i, 1)], wsems[slot])

    def wait_write(i, slot):
        pltpu.make_async_copy(
            bufs[slot], out_hbm.at[pl.ds(base + i, 1)], wsems[slot]
        ).wait()

    for i in range(AHEAD):
        fire_gather(i, i % NBUF)

    def step_static(i):
        f = i + AHEAD
        if f <= RPW - 1:
            if f - NBUF >= 0:
                wait_write(f - NBUF, f % NBUF)
            fire_gather(f, f % NBUF)
        wait_gather(i, i % NBUF)
        fire_write(i, i % NBUF)

    for i in range(NBUF):
        step_static(i)

    def body(g, carry):
        for k in range(NBUF):
            i = NBUF * g + k
            fslot = (k + AHEAD) % NBUF
            wait_write(i + AHEAD - NBUF, fslot)
            fire_gather(i + AHEAD, fslot)
            wait_gather(i, k)
            fire_write(i, k)
        return carry

    n_blocks = (RPW - AHEAD) // NBUF
    lax.fori_loop(1, n_blocks, body, 0)

    for i in range(NBUF * n_blocks, RPW):
        step_static(i)
    for i in range(RPW - NBUF, RPW):
        wait_write(i, i % NBUF)


def kernel(r_idx, qkv):
    gidx = (
        jnp.arange(N, dtype=jnp.int32)[:, None, None] * P3
        + r_idx.astype(jnp.int32)
    ).reshape(NW, RPW, 1)
    table = qkv.reshape(ROWS, W3, CKV)
    out = _sc_gather(gidx, table)
    return out.reshape(N, P3, TOPK, W3, CKV)


# speedup vs baseline: 48.4939x; 1.2446x over previous
import functools

import jax
import jax.numpy as jnp
from jax import lax
from jax.experimental import pallas as pl
from jax.experimental.pallas import tpu as pltpu
from jax.experimental.pallas import tpu_sc as plsc

N, P3, W3, CKV = 8, 49, 64, 384
TOPK = 4
ROWS = N * P3
B = N * P3 * TOPK
NC, NS = 2, 16
NW = NC * NS
NQ = 4
WS = W3 // NQ
RPB = P3 * TOPK
PLPAD = 224
OFFPAD = 80
NBUF = 7
L = 16

_mesh = plsc.VectorSubcoreMesh(core_axis_name="c", subcore_axis_name="s")


@functools.partial(
    pl.kernel,
    mesh=_mesh,
    out_type=jax.ShapeDtypeStruct((B * NQ, WS, CKV), jnp.float32),
    scratch_types=[
        pltpu.VMEM((PLPAD,), jnp.int32),
        pltpu.VMEM((OFFPAD,), jnp.int32),
    ]
    + [pltpu.VMEM((1, WS, CKV), jnp.float32) for _ in range(NBUF)]
    + [pltpu.SemaphoreType.DMA for _ in range(2 * NBUF)],
)
def _sc_gather(plist_hbm, off_hbm, table_hbm, out_hbm, plv, offv, *scr):
    bufs = scr[:NBUF]
    gsems = scr[NBUF : 2 * NBUF]
    wsems = scr[2 * NBUF :]
    wid = lax.axis_index("s") * NC + lax.axis_index("c")
    b = wid // NQ
    q = wid % NQ
    sbase = b * RPB + q
    dbase = b * RPB * NQ + q
    pltpu.sync_copy(plist_hbm.at[b], plv)
    pltpu.sync_copy(off_hbm.at[b], offv)

    def wait_one_write(u):
        pltpu.make_async_copy(bufs[u], out_hbm.at[pl.ds(0, 1)], wsems[u]).wait()

    def region(j, u, cnt_u):
        o_j = offv[pl.ds(j, L)][0]
        c_j = offv[pl.ds(j + 1, L)][0] - o_j
        srow = sbase + j * NQ

        lax.fori_loop(0, cnt_u, lambda t, c: (wait_one_write(u), c)[1], 0)
        pltpu.async_copy(table_hbm.at[pl.ds(srow, 1)], bufs[u], gsems[u])
        pltpu.make_async_copy(
            table_hbm.at[pl.ds(srow, 1)], bufs[u], gsems[u]
        ).wait()

        def wbody(t, c):
            p = plv[pl.ds(o_j + t, L)][0]
            pltpu.async_copy(
                bufs[u], out_hbm.at[pl.ds(dbase + p * NQ, 1)], wsems[u]
            )
            return c

        lax.fori_loop(0, c_j, wbody, 0)
        return c_j

    def block(g, carry):
        cnts = list(carry)
        for u in range(NBUF):
            cnts[u] = region(NBUF * g + u, u, cnts[u])
        return tuple(cnts)

    carry = lax.fori_loop(0, P3 // NBUF, block, (jnp.int32(0),) * NBUF)

    for u in range(NBUF):
        lax.fori_loop(0, carry[u], lambda t, c, u=u: (wait_one_write(u), c)[1], 0)


def kernel(r_idx, qkv):
    ridx = r_idx.reshape(N, RPB).astype(jnp.int32)
    order = jnp.argsort(ridx, axis=1, stable=True).astype(jnp.int32)
    counts = jnp.sum(
        ridx[:, :, None] == jnp.arange(P3, dtype=jnp.int32)[None, None, :],
        axis=1,
        dtype=jnp.int32,
    )
    offsets = jnp.concatenate(
        [jnp.zeros((N, 1), jnp.int32), jnp.cumsum(counts, axis=1, dtype=jnp.int32)],
        axis=1,
    )
    plist = jnp.pad(order, ((0, 0), (0, PLPAD - RPB)))
    offs = jnp.pad(offsets, ((0, 0), (0, OFFPAD - (P3 + 1))))
    table = qkv.reshape(ROWS * NQ, WS, CKV)
    out = _sc_gather(plist, offs, table)
    return out.reshape(N, P3, TOPK, W3, CKV)


# speedup vs baseline: 49.6212x; 1.0232x over previous
import functools

import jax
import jax.numpy as jnp
from jax import lax
from jax.experimental import pallas as pl
from jax.experimental.pallas import tpu as pltpu
from jax.experimental.pallas import tpu_sc as plsc

N, P3, W3, CKV = 8, 49, 64, 384
TOPK = 4
ROWS = N * P3
B = N * P3 * TOPK
NC, NS = 2, 16
NW = NC * NS
NQ = 4
WS = W3 // NQ
RPB = P3 * TOPK
PLPAD = 224
OFFPAD = 80
NBUF = 7
L = 16

_mesh = plsc.VectorSubcoreMesh(core_axis_name="c", subcore_axis_name="s")


@functools.partial(
    pl.kernel,
    mesh=_mesh,
    out_type=jax.ShapeDtypeStruct((B * NQ, WS, CKV), jnp.float32),
    scratch_types=[
        pltpu.VMEM((PLPAD,), jnp.int32),
        pltpu.VMEM((OFFPAD,), jnp.int32),
    ]
    + [pltpu.VMEM((1, WS, CKV), jnp.float32) for _ in range(NBUF)]
    + [pltpu.SemaphoreType.DMA for _ in range(2 * NBUF)],
)
def _sc_gather(plist_hbm, off_hbm, table_hbm, out_hbm, plv, offv, *scr):
    bufs = scr[:NBUF]
    gsems = scr[NBUF : 2 * NBUF]
    wsems = scr[2 * NBUF :]
    wid = lax.axis_index("s") * NC + lax.axis_index("c")
    b = wid // NQ
    q = wid % NQ
    sbase = b * RPB + q
    dbase = b * RPB * NQ + q
    pltpu.sync_copy(plist_hbm.at[b], plv)
    pltpu.sync_copy(off_hbm.at[b], offv)

    def wait_one_write(u):
        pltpu.make_async_copy(bufs[u], out_hbm.at[pl.ds(0, 1)], wsems[u]).wait()

    def fire_gather(j, u):
        pltpu.async_copy(
            table_hbm.at[pl.ds(sbase + j * NQ, 1)], bufs[u], gsems[u]
        )

    def wait_gather(u):
        pltpu.make_async_copy(
            table_hbm.at[pl.ds(sbase, 1)], bufs[u], gsems[u]
        ).wait()

    def region(j, u, cnt_u1, last):
        o_j = offv[pl.ds(j, L)][0]
        c_j = offv[pl.ds(j + 1, L)][0] - o_j
        wait_gather(u)
        if not last:
            u1 = (u + 1) % NBUF
            lax.fori_loop(0, cnt_u1, lambda t, c: (wait_one_write(u1), c)[1], 0)
            fire_gather(j + 1, u1)

        def wbody(t, c):
            p = plv[pl.ds(o_j + t, L)][0]
            pltpu.async_copy(
                bufs[u], out_hbm.at[pl.ds(dbase + p * NQ, 1)], wsems[u]
            )
            return c

        lax.fori_loop(0, c_j, wbody, 0)
        return c_j

    fire_gather(0, 0)

    def block(g, carry):
        cnts = list(carry)
        for u in range(NBUF):
            cnts[u] = region(NBUF * g + u, u, cnts[(u + 1) % NBUF], False)
        return tuple(cnts)

    carry = lax.fori_loop(0, P3 // NBUF - 1, block, (jnp.int32(0),) * NBUF)

    cnts = list(carry)
    for u in range(NBUF):
        j = P3 - NBUF + u
        cnts[u] = region(j, u, cnts[(u + 1) % NBUF], j == P3 - 1)

    for u in range(NBUF):
        lax.fori_loop(0, cnts[u], lambda t, c, u=u: (wait_one_write(u), c)[1], 0)


def kernel(r_idx, qkv):
    ridx = r_idx.reshape(N, RPB).astype(jnp.int32)
    order = jnp.argsort(ridx, axis=1, stable=True).astype(jnp.int32)
    counts = jnp.sum(
        ridx[:, :, None] == jnp.arange(P3, dtype=jnp.int32)[None, None, :],
        axis=1,
        dtype=jnp.int32,
    )
    offsets = jnp.concatenate(
        [jnp.zeros((N, 1), jnp.int32), jnp.cumsum(counts, axis=1, dtype=jnp.int32)],
        axis=1,
    )
    plist = jnp.pad(order, ((0, 0), (0, PLPAD - RPB)))
    offs = jnp.pad(offsets, ((0, 0), (0, OFFPAD - (P3 + 1))))
    table = qkv.reshape(ROWS * NQ, WS, CKV)
    out = _sc_gather(plist, offs, table)
    return out.reshape(N, P3, TOPK, W3, CKV)


# speedup vs baseline: 50.4481x; 1.0167x over previous
import functools

import jax
import jax.numpy as jnp
from jax import lax
from jax.experimental import pallas as pl
from jax.experimental.pallas import tpu as pltpu
from jax.experimental.pallas import tpu_sc as plsc

N, P3, W3, CKV = 8, 49, 64, 384
TOPK = 4
ROWS = N * P3
B = N * P3 * TOPK
NC, NS = 2, 16
NW = NC * NS
NQ = 4
WS = W3 // NQ
RPB = P3 * TOPK
PLPAD = 224
OFFPAD = 80
NBUF = 7
L = 16

_mesh = plsc.VectorSubcoreMesh(core_axis_name="c", subcore_axis_name="s")


@functools.partial(
    pl.kernel,
    mesh=_mesh,
    out_type=jax.ShapeDtypeStruct((B * NQ, WS, CKV), jnp.float32),
    scratch_types=[
        pltpu.VMEM((PLPAD,), jnp.int32),
        pltpu.VMEM((OFFPAD,), jnp.int32),
    ]
    + [pltpu.VMEM((1, WS, CKV), jnp.float32) for _ in range(NBUF)]
    + [pltpu.SemaphoreType.DMA for _ in range(2 * NBUF)],
)
def _sc_gather(plist_hbm, off_hbm, table_hbm, out_hbm, plv, offv, *scr):
    bufs = scr[:NBUF]
    gsems = scr[NBUF : 2 * NBUF]
    wsems = scr[2 * NBUF :]
    wid = lax.axis_index("s") * NC + lax.axis_index("c")
    b = wid // NQ
    q = wid % NQ
    sbase = b * RPB + q
    dbase = b * RPB * NQ + q
    pltpu.sync_copy(plist_hbm.at[b], plv)
    pltpu.sync_copy(off_hbm.at[b], offv)

    def wait_one_write(u):
        pltpu.make_async_copy(bufs[u], out_hbm.at[pl.ds(0, 1)], wsems[u]).wait()

    def fire_gather(j, u):
        pltpu.async_copy(
            table_hbm.at[pl.ds(sbase + j * NQ, 1)], bufs[u], gsems[u]
        )

    def wait_gather(u):
        pltpu.make_async_copy(
            table_hbm.at[pl.ds(sbase, 1)], bufs[u], gsems[u]
        ).wait()

    def region(j, u, cnt_u2, last):
        o_j = offv[pl.ds(j, L)][0]
        c_j = offv[pl.ds(j + 1, L)][0] - o_j
        wait_gather(u)
        if not last:
            u2 = (u + 2) % NBUF
            lax.fori_loop(0, cnt_u2, lambda t, c: (wait_one_write(u2), c)[1], 0)
            fire_gather(j + 2, u2)

        def wbody(t, c):
            p = plv[pl.ds(o_j + t, L)][0]
            pltpu.async_copy(
                bufs[u], out_hbm.at[pl.ds(dbase + p * NQ, 1)], wsems[u]
            )
            return c

        lax.fori_loop(0, c_j, wbody, 0)
        return c_j

    fire_gather(0, 0)
    fire_gather(1, 1)

    def block(g, carry):
        cnts = list(carry)
        for u in range(NBUF):
            cnts[u] = region(NBUF * g + u, u, cnts[(u + 2) % NBUF], False)
        return tuple(cnts)

    carry = lax.fori_loop(0, P3 // NBUF - 1, block, (jnp.int32(0),) * NBUF)

    cnts = list(carry)
    for u in range(NBUF):
        j = P3 - NBUF + u
        cnts[u] = region(j, u, cnts[(u + 2) % NBUF], j >= P3 - 2)

    for u in range(NBUF):
        lax.fori_loop(0, cnts[u], lambda t, c, u=u: (wait_one_write(u), c)[1], 0)


def kernel(r_idx, qkv):
    ridx = r_idx.reshape(N, RPB).astype(jnp.int32)
    order = jnp.argsort(ridx, axis=1, stable=True).astype(jnp.int32)
    counts = jnp.sum(
        ridx[:, :, None] == jnp.arange(P3, dtype=jnp.int32)[None, None, :],
        axis=1,
        dtype=jnp.int32,
    )
    offsets = jnp.concatenate(
        [jnp.zeros((N, 1), jnp.int32), jnp.cumsum(counts, axis=1, dtype=jnp.int32)],
        axis=1,
    )
    plist = jnp.pad(order, ((0, 0), (0, PLPAD - RPB)))
    offs = jnp.pad(offsets, ((0, 0), (0, OFFPAD - (P3 + 1))))
    table = qkv.reshape(ROWS * NQ, WS, CKV)
    out = _sc_gather(plist, offs, table)
    return out.reshape(N, P3, TOPK, W3, CKV)
